# Initial kernel scaffold; baseline (speedup 1.0000x reference)
import jax
import jax.numpy as jnp
from jax.experimental import pallas as pl


def kernel(x, Wg, Wfc, Wproj):
    raise NotImplementedError("write your pallas kernel here")


# speedup vs baseline: 1.8170x; 1.8170x over previous
import jax
import jax.numpy as jnp
from jax.experimental import pallas as pl
from jax.experimental.pallas import tpu as pltpu

D = 768
E = 16
N = 16
F = 4 * D
C = 1536
NC = F // C


def _moe_body(x_ref, wg_ref, wfc_ref, wproj_ref, out_ref, g_ref):
    i = pl.program_id(0)
    c = pl.program_id(1)

    @pl.when((i == 0) & (c == 0))
    def _gating():
        xf = x_ref[:]
        logits = jnp.dot(xf, wg_ref[:],
                         preferred_element_type=jnp.float32)
        m = jnp.max(logits, axis=-1, keepdims=True)
        ex = jnp.exp(logits - m)
        gates = ex / jnp.sum(ex, axis=-1, keepdims=True)
        gk = gates[:, :, None]
        gm = gates[:, None, :]
        iota_k = jax.lax.broadcasted_iota(jnp.int32, (N, E, E), 1)
        iota_m = jax.lax.broadcasted_iota(jnp.int32, (N, E, E), 2)
        before = (gm > gk) | ((gm == gk) & (iota_m < iota_k))
        rank = jnp.sum(before.astype(jnp.int32), axis=2)
        onehot = (rank[:, :, None]
                  == jax.lax.broadcasted_iota(jnp.int32, (N, E, E), 2))
        srt = jnp.sum(gates[:, :, None] * onehot.astype(jnp.float32), axis=1)
        srt = srt / jnp.sum(srt, axis=-1, keepdims=True)
        g_ref[:] = srt

    h = jnp.dot(x_ref[:], wfc_ref[0], preferred_element_type=jnp.float32)
    a = 0.5 * h * (1.0 + jax.lax.erf(h * 0.7071067811865476))

    grow = g_ref[pl.ds(i, 1), :]
    z = jnp.dot(grow, a, preferred_element_type=jnp.float32)
    part = jnp.dot(z, wproj_ref[0], preferred_element_type=jnp.float32)

    @pl.when(c == 0)
    def _init():
        out_ref[0] = part

    @pl.when(c != 0)
    def _acc():
        out_ref[0] += part


def kernel(x, Wg, Wfc, Wproj):
    orig_shape = x.shape
    xf = x.reshape(-1, D)
    out = pl.pallas_call(
        _moe_body,
        grid=(E, NC),
        in_specs=[
            pl.BlockSpec((N, D), lambda i, c: (0, 0)),
            pl.BlockSpec((D, E), lambda i, c: (0, 0)),
            pl.BlockSpec((1, D, C), lambda i, c: (i, 0, c)),
            pl.BlockSpec((1, C, D), lambda i, c: (i, c, 0)),
        ],
        out_specs=pl.BlockSpec((1, 1, D), lambda i, c: (i, 0, 0)),
        out_shape=jax.ShapeDtypeStruct((E, 1, D), jnp.float32),
        scratch_shapes=[pltpu.VMEM((N, E), jnp.float32)],
        compiler_params=pltpu.CompilerParams(
            dimension_semantics=("arbitrary", "arbitrary"),
        ),
    )(xf, Wg, Wfc, Wproj)
    return out.reshape(orig_shape)
